# baseline (speedup 1.0000x reference)
import functools

import jax
import jax.numpy as jnp
from jax import lax
from jax.experimental import pallas as pl
from jax.experimental.pallas import tpu as pltpu
from jax.experimental.pallas import tpu_sc as plsc

VOCAB = 1_000_000
HIDDEN = 64
BATCH = 4096
HIST = 200

_NW = 32
_ROWS_PER_W = BATCH // _NW


def _make_gather():
  mesh = plsc.VectorSubcoreMesh(core_axis_name="c", subcore_axis_name="s")

  @functools.partial(
      pl.kernel,
      out_type=jax.ShapeDtypeStruct((BATCH, HIST, HIDDEN), jnp.float32),
      mesh=mesh,
      scratch_types=[
          pltpu.VMEM((_ROWS_PER_W, HIST), jnp.int32),
          pltpu.VMEM((HIST, HIDDEN), jnp.float32),
          pltpu.VMEM((HIST, HIDDEN), jnp.float32),
          pltpu.SemaphoreType.DMA,
          pltpu.SemaphoreType.DMA,
          pltpu.SemaphoreType.DMA,
          pltpu.SemaphoreType.DMA,
      ],
      compiler_params=pltpu.CompilerParams(use_tc_tiling_on_sc=False),
  )
  def gather_kernel(emb_hbm, idx_hbm, out_hbm, idx_v, rows0, rows1,
                    gsem0, gsem1, ssem0, ssem1):
    wid = lax.axis_index("s") * 2 + lax.axis_index("c")
    base = wid * _ROWS_PER_W
    pltpu.sync_copy(idx_hbm.at[pl.ds(base, _ROWS_PER_W)], idx_v)

    rows = (rows0, rows1)
    gsem = (gsem0, gsem1)
    ssem = (ssem0, ssem1)

    def gather(i, b):
      pltpu.make_async_copy(
          emb_hbm.at[idx_v.at[i]], rows[b], gsem[b]).start()

    def gather_wait(b):
      pltpu.make_async_copy(
          emb_hbm.at[idx_v.at[0]], rows[b], gsem[b]).wait()

    def store(i, b):
      pltpu.make_async_copy(rows[b], out_hbm.at[base + i], ssem[b]).start()

    def store_wait(b):
      pltpu.make_async_copy(rows[b], out_hbm.at[base], ssem[b]).wait()

    gather(0, 0)
    gather(1, 1)
    gather_wait(0)
    store(0, 0)

    def body(k, _):
      g = 2 + 2 * k
      for b in range(2):
        i = g + b
        store_wait(b)
        gather(i, b)
        gather_wait(1 - b)
        store(i - 1, 1 - b)
      return 0

    lax.fori_loop(0, (_ROWS_PER_W - 2) // 2, body, 0, unroll=False)

    gather_wait(1)
    store(_ROWS_PER_W - 1, 1)
    store_wait(0)
    store_wait(1)

  return gather_kernel


_gather = _make_gather()


def kernel(input_ids, emb):
  return _gather(emb, input_ids.astype(jnp.int32))
